# baseline (speedup 1.0000x reference)
import jax
import jax.numpy as jnp
from jax import lax
from jax.experimental import pallas as pl
from jax.experimental.pallas import tpu as pltpu
from jax.experimental.pallas import tpu_sc as plsc

_GM = 3.5
_B = 16384
_L = 50
_D = 64
_LANES = 16
_NC = 2
_NS = 16
_NW = _NC * _NS
_QPW = _B // _NW
_PAIR = 2
_IDXW = _PAIR * _L
_NPAIR = _QPW // _PAIR
_NBUF = 4
_GPG = 8
_NGRP = _QPW // _LANES
_QCHUNK = 128
_NQC = _QPW // _QCHUNK
_RCH = 5
_SCALE = 262144.0
_INV_SCALE2 = 1.0 / (_SCALE * _SCALE)


def _sc_body(uid_hbm, iid_hbm, rated_hbm, isq_hbm, P_hbm, Q_hbm, BU_hbm,
             BI_hbm, Y_hbm, out_hbm,
             uidv, iidv, idxv, isqv, pv, qv, buv, biv, outv,
             yb0, yb1, yb2, yb3, sem_pq, ys0, ys1, ys2, ys3):
    ybufs = (yb0, yb1, yb2, yb3)
    ysems = (ys0, ys1, ys2, ys3)
    wid = lax.axis_index("s") * _NC + lax.axis_index("c")
    qbase = wid * _QPW

    pltpu.sync_copy(rated_hbm.at[pl.ds(wid * _NPAIR, _NPAIR)], idxv)
    pltpu.sync_copy(uid_hbm.at[pl.ds(qbase, _QPW)], uidv)
    pltpu.sync_copy(iid_hbm.at[pl.ds(qbase, _QPW)], iidv)
    pltpu.sync_copy(isq_hbm.at[pl.ds(qbase, _QPW)], isqv)

    for k in range(_NQC):
        sl = pl.ds(k * _QCHUNK, _QCHUNK)
        pltpu.async_copy(P_hbm.at[uidv.at[sl]], pv.at[sl], sem_pq)
        pltpu.async_copy(Q_hbm.at[iidv.at[sl]], qv.at[sl], sem_pq)
        pltpu.async_copy(BU_hbm.at[uidv.at[sl]], buv.at[sl], sem_pq)
        pltpu.async_copy(BI_hbm.at[iidv.at[sl]], biv.at[sl], sem_pq)

    for b in range(_NBUF):
        pltpu.async_copy(Y_hbm.at[idxv.at[b]], ybufs[b], ysems[b])

    for k in range(_NQC):
        sl = pl.ds(k * _QCHUNK, _QCHUNK)
        pltpu.make_async_copy(P_hbm.at[uidv.at[sl]], pv.at[sl], sem_pq).wait()
        pltpu.make_async_copy(Q_hbm.at[iidv.at[sl]], qv.at[sl], sem_pq).wait()
        pltpu.make_async_copy(BU_hbm.at[uidv.at[sl]], buv.at[sl],
                              sem_pq).wait()
        pltpu.make_async_copy(BI_hbm.at[iidv.at[sl]], biv.at[sl],
                              sem_pq).wait()

    zeros = jnp.zeros((_LANES,), jnp.float32)
    lane_iota = lax.iota(jnp.int32, _LANES)

    def lane_sum(v):
        for k in range(4):
            perm = lane_iota ^ (1 << k)
            v = v + v.at[perm].get(mode="promise_in_bounds")
        return v

    def row4(ref, r):
        out = []
        for h in range(2):
            v = ref[r, pl.ds(h * _LANES, _LANES)]
            out.append(((v << 16) >> 16).astype(jnp.float32))
            out.append((v >> 16).astype(jnp.float32))
        return tuple(out)

    def bag_sum(ybuf, row0):
        def body(j, acc):
            a = list(acc)
            base = row0 + j * _RCH
            for u in range(_RCH):
                y = row4(ybuf, base + u)
                for dd in range(4):
                    a[dd] = a[dd] + y[dd]
            return tuple(a)
        return lax.fori_loop(0, _L // _RCH, body, (zeros,) * 4)

    def group(grp, refires):
        q16 = grp * _LANES
        gbase = grp * _GPG
        sv = isqv[pl.ds(q16, _LANES)]
        pred = zeros
        for j in range(_GPG):
            g = gbase + j
            b = j % _NBUF
            pltpu.make_async_copy(Y_hbm.at[idxv.at[g]], ybufs[b],
                                  ysems[b]).wait()
            for t in range(_PAIR):
                lane = _PAIR * j + t
                acc = bag_sum(ybufs[b], t * _L)
                lane_ix = jnp.full((_LANES,), lane, jnp.int32)
                s = sv.at[lane_ix].get(mode="promise_in_bounds")
                ql = q16 + lane
                pr = row4(pv, ql)
                qr = row4(qv, ql)
                dot = zeros
                for dd in range(4):
                    dot = dot + (pr[dd] + s * acc[dd]) * qr[dd]
                pred = pred + jnp.where(lane_iota == lane, lane_sum(dot),
                                        zeros)
            if refires[j]:
                pltpu.async_copy(Y_hbm.at[idxv.at[g + _NBUF]], ybufs[b],
                                 ysems[b])
        pred = (pred * _INV_SCALE2 + buv[pl.ds(q16, _LANES)]
                + biv[pl.ds(q16, _LANES)] + _GM)
        outv[pl.ds(q16, _LANES)] = pred

    def main_body(i, carry):
        group(i, (True,) * _GPG)
        return carry

    lax.fori_loop(0, _NGRP - 1, main_body, 0)
    group(_NGRP - 1, (True,) * _NBUF + (False,) * (_GPG - _NBUF))

    pltpu.sync_copy(outv, out_hbm.at[pl.ds(qbase, _QPW)])


def kernel(user_id, item_id, rated_items, rated_counts, P, Q, B_U, B_I, Y):
    isq = lax.rsqrt(rated_counts.astype(jnp.float32))
    rated2 = rated_items.astype(jnp.int32).reshape(_B // _PAIR, _IDXW)
    bu1 = B_U.reshape(-1)
    bi1 = B_I.reshape(-1)
    def _pack_i16(t):
        ti = jnp.clip(jnp.round(t * _SCALE), -32767.0, 32767.0)
        ti = ti.astype(jnp.int16).reshape(t.shape[0], t.shape[1] // 2, 2)
        return lax.bitcast_convert_type(ti, jnp.int32)

    Pb = _pack_i16(P)
    Qb = _pack_i16(Q)
    Yb = _pack_i16(Y)
    mesh = plsc.VectorSubcoreMesh(core_axis_name="c", subcore_axis_name="s")
    run = pl.kernel(
        _sc_body,
        mesh=mesh,
        compiler_params=pltpu.CompilerParams(use_tc_tiling_on_sc=False),
        out_type=jax.ShapeDtypeStruct((_B,), jnp.float32),
        scratch_types=[
            pltpu.VMEM((_QPW,), jnp.int32),
            pltpu.VMEM((_QPW,), jnp.int32),
            pltpu.VMEM((_NPAIR, _IDXW), jnp.int32),
            pltpu.VMEM((_QPW,), jnp.float32),
            pltpu.VMEM((_QPW, _D // 2), jnp.int32),
            pltpu.VMEM((_QPW, _D // 2), jnp.int32),
            pltpu.VMEM((_QPW,), jnp.float32),
            pltpu.VMEM((_QPW,), jnp.float32),
            pltpu.VMEM((_QPW,), jnp.float32),
            pltpu.VMEM((_IDXW, _D // 2), jnp.int32),
            pltpu.VMEM((_IDXW, _D // 2), jnp.int32),
            pltpu.VMEM((_IDXW, _D // 2), jnp.int32),
            pltpu.VMEM((_IDXW, _D // 2), jnp.int32),
            pltpu.SemaphoreType.DMA,
            pltpu.SemaphoreType.DMA,
            pltpu.SemaphoreType.DMA,
            pltpu.SemaphoreType.DMA,
            pltpu.SemaphoreType.DMA,
        ],
    )
    return run(user_id.astype(jnp.int32), item_id.astype(jnp.int32),
               rated2, isq, Pb, Qb, bu1, bi1, Yb)


# speedup vs baseline: 3.0373x; 3.0373x over previous
import jax
import jax.numpy as jnp
from jax import lax
from jax.experimental import pallas as pl
from jax.experimental.pallas import tpu as pltpu
from jax.experimental.pallas import tpu_sc as plsc

_GM = 3.5
_B = 16384
_L = 50
_D = 64
_LANES = 16
_NC = 2
_NS = 16
_NW = _NC * _NS
_QPW = _B // _NW
_PAIR = 2
_IDXW = _PAIR * _L
_NPAIR = _QPW // _PAIR
_NBUF = 4
_GPG = 8
_NGRP = _QPW // _LANES
_QCHUNK = 128
_NQC = _QPW // _QCHUNK
_RCH = 5


def _sc_body(uid_hbm, iid_hbm, rated_hbm, isq_hbm, P_hbm, Q_hbm, BU_hbm,
             BI_hbm, Y_hbm, out_hbm,
             uidv, iidv, idxv, isqv, pv, qv, buv, biv, outv,
             yb0, yb1, yb2, yb3, sem_pq, ys0, ys1, ys2, ys3):
    ybufs = (yb0, yb1, yb2, yb3)
    ysems = (ys0, ys1, ys2, ys3)
    wid = lax.axis_index("s") * _NC + lax.axis_index("c")
    qbase = wid * _QPW

    pltpu.sync_copy(rated_hbm.at[pl.ds(wid * _NPAIR, _NPAIR)], idxv)
    pltpu.sync_copy(uid_hbm.at[pl.ds(qbase, _QPW)], uidv)
    pltpu.sync_copy(iid_hbm.at[pl.ds(qbase, _QPW)], iidv)
    pltpu.sync_copy(isq_hbm.at[pl.ds(qbase, _QPW)], isqv)

    for k in range(_NQC):
        sl = pl.ds(k * _QCHUNK, _QCHUNK)
        pltpu.async_copy(P_hbm.at[uidv.at[sl]], pv.at[sl], sem_pq)
        pltpu.async_copy(Q_hbm.at[iidv.at[sl]], qv.at[sl], sem_pq)
        pltpu.async_copy(BU_hbm.at[uidv.at[sl]], buv.at[sl], sem_pq)
        pltpu.async_copy(BI_hbm.at[iidv.at[sl]], biv.at[sl], sem_pq)

    for b in range(_NBUF):
        pltpu.async_copy(Y_hbm.at[idxv.at[b]], ybufs[b], ysems[b])

    for k in range(_NQC):
        sl = pl.ds(k * _QCHUNK, _QCHUNK)
        pltpu.make_async_copy(P_hbm.at[uidv.at[sl]], pv.at[sl], sem_pq).wait()
        pltpu.make_async_copy(Q_hbm.at[iidv.at[sl]], qv.at[sl], sem_pq).wait()
        pltpu.make_async_copy(BU_hbm.at[uidv.at[sl]], buv.at[sl],
                              sem_pq).wait()
        pltpu.make_async_copy(BI_hbm.at[iidv.at[sl]], biv.at[sl],
                              sem_pq).wait()

    zeros = jnp.zeros((_LANES,), jnp.float32)
    lane_iota = lax.iota(jnp.int32, _LANES)

    def lane_sum(v):
        for k in range(4):
            perm = lane_iota ^ (1 << k)
            v = v + v.at[perm].get(mode="promise_in_bounds")
        return v

    def row4(ref, r):
        return tuple(ref[r, pl.ds(dd * _LANES, _LANES)] for dd in range(4))

    def bag_sum(ybuf, row0):
        def body(j, acc):
            a = list(acc)
            base = row0 + j * _RCH
            for u in range(_RCH):
                y = row4(ybuf, base + u)
                for dd in range(4):
                    a[dd] = a[dd] + y[dd]
            return tuple(a)
        return lax.fori_loop(0, _L // _RCH, body, (zeros,) * 4)

    def group(grp, refires):
        q16 = grp * _LANES
        gbase = grp * _GPG
        sv = isqv[pl.ds(q16, _LANES)]
        pred = zeros
        for j in range(_GPG):
            g = gbase + j
            b = j % _NBUF
            pltpu.make_async_copy(Y_hbm.at[idxv.at[g]], ybufs[b],
                                  ysems[b]).wait()
            for t in range(_PAIR):
                lane = _PAIR * j + t
                acc = bag_sum(ybufs[b], t * _L)
                lane_ix = jnp.full((_LANES,), lane, jnp.int32)
                s = sv.at[lane_ix].get(mode="promise_in_bounds")
                ql = q16 + lane
                pr = row4(pv, ql)
                qr = row4(qv, ql)
                dot = zeros
                for dd in range(4):
                    dot = dot + (pr[dd] + s * acc[dd]) * qr[dd]
                pred = pred + jnp.where(lane_iota == lane, lane_sum(dot),
                                        zeros)
            if refires[j]:
                pltpu.async_copy(Y_hbm.at[idxv.at[g + _NBUF]], ybufs[b],
                                 ysems[b])
        pred = pred + buv[pl.ds(q16, _LANES)] + biv[pl.ds(q16, _LANES)] + _GM
        outv[pl.ds(q16, _LANES)] = pred

    def main_body(i, carry):
        group(i, (True,) * _GPG)
        return carry

    lax.fori_loop(0, _NGRP - 1, main_body, 0)
    group(_NGRP - 1, (True,) * _NBUF + (False,) * (_GPG - _NBUF))

    pltpu.sync_copy(outv, out_hbm.at[pl.ds(qbase, _QPW)])


def kernel(user_id, item_id, rated_items, rated_counts, P, Q, B_U, B_I, Y):
    isq = lax.rsqrt(rated_counts.astype(jnp.float32))
    rated2 = rated_items.astype(jnp.int32).reshape(_B // _PAIR, _IDXW)
    bu1 = B_U.reshape(-1)
    bi1 = B_I.reshape(-1)
    mesh = plsc.VectorSubcoreMesh(core_axis_name="c", subcore_axis_name="s")
    run = pl.kernel(
        _sc_body,
        mesh=mesh,
        compiler_params=pltpu.CompilerParams(use_tc_tiling_on_sc=False),
        out_type=jax.ShapeDtypeStruct((_B,), jnp.float32),
        scratch_types=[
            pltpu.VMEM((_QPW,), jnp.int32),
            pltpu.VMEM((_QPW,), jnp.int32),
            pltpu.VMEM((_NPAIR, _IDXW), jnp.int32),
            pltpu.VMEM((_QPW,), jnp.float32),
            pltpu.VMEM((_QPW, _D), jnp.float32),
            pltpu.VMEM((_QPW, _D), jnp.float32),
            pltpu.VMEM((_QPW,), jnp.float32),
            pltpu.VMEM((_QPW,), jnp.float32),
            pltpu.VMEM((_QPW,), jnp.float32),
            pltpu.VMEM((_IDXW, _D), jnp.float32),
            pltpu.VMEM((_IDXW, _D), jnp.float32),
            pltpu.VMEM((_IDXW, _D), jnp.float32),
            pltpu.VMEM((_IDXW, _D), jnp.float32),
            pltpu.SemaphoreType.DMA,
            pltpu.SemaphoreType.DMA,
            pltpu.SemaphoreType.DMA,
            pltpu.SemaphoreType.DMA,
            pltpu.SemaphoreType.DMA,
        ],
    )
    return run(user_id.astype(jnp.int32), item_id.astype(jnp.int32),
               rated2, isq, P, Q, bu1, bi1, Y)
